# baseline (speedup 1.0000x reference)
import functools

import jax
import jax.numpy as jnp
from jax import lax
from jax.experimental import pallas as pl
from jax.experimental.pallas import tpu as pltpu
from jax.experimental.pallas import tpu_sc as plsc

NC = 2
NS = 16
CH = 128


def _sc_mesh():
    return plsc.VectorSubcoreMesh(core_axis_name="c", subcore_axis_name="s")


def _make_deg_kernel(n_pad, e_pad):
    chunks_per_tile = e_pad // (NC * NS * CH)
    rows_per_tile = n_pad // NS

    @functools.partial(
        pl.kernel,
        mesh=_sc_mesh(),
        out_type=jax.ShapeDtypeStruct((NC * n_pad,), jnp.float32),
        scratch_types=[
            pltpu.VMEM((CH,), jnp.int32),
            pltpu.VMEM((CH,), jnp.float32),
            pltpu.VMEM((rows_per_tile,), jnp.float32),
            pltpu.VMEM_SHARED((n_pad,), jnp.float32),
            pltpu.SemaphoreType.DMA,
        ],
    )
    def deg_kernel(dst_hbm, ones_hbm, zeros_hbm, out_hbm,
                   dst_v, ones_v, row_v, deg_sh, sem):
        c = lax.axis_index("c")
        s = lax.axis_index("s")
        row0 = s * rows_per_tile

        @pl.when(c == 0)
        def _():
            pltpu.sync_copy(ones_hbm.at[pl.ds(row0, rows_per_tile)], row_v)

        @pl.when(c != 0)
        def _():
            pltpu.sync_copy(zeros_hbm.at[pl.ds(row0, rows_per_tile)], row_v)

        pltpu.sync_copy(row_v, deg_sh.at[pl.ds(row0, rows_per_tile)])
        pltpu.sync_copy(ones_hbm.at[pl.ds(0, CH)], ones_v)
        plsc.subcore_barrier()

        base_chunk = (c * NS + s) * chunks_per_tile

        @pl.loop(0, chunks_per_tile)
        def _(j):
            off = (base_chunk + j) * CH
            pltpu.sync_copy(dst_hbm.at[pl.ds(off, CH)], dst_v)
            pltpu.sync_copy(ones_v, deg_sh.at[dst_v], add=True)

        plsc.subcore_barrier()

        pltpu.sync_copy(deg_sh.at[pl.ds(row0, rows_per_tile)], row_v)
        pltpu.sync_copy(row_v, out_hbm.at[pl.ds(c * n_pad + row0, rows_per_tile)])

    return deg_kernel


IDX_Q = 8


CPT_FRAC0 = 88 / 158


def _make_agg_kernel(n_pad, e_pad, dim):
    chunks_total = e_pad // (NS * CH)
    cpt0 = 2 * int(round(chunks_total * CPT_FRAC0 / 2))
    cpt1 = chunks_total - cpt0
    assert cpt1 % 2 == 0 and cpt0 >= 2 and cpt1 >= 2
    rows_per_tile = n_pad // NS
    n_init_chunks = (rows_per_tile + CH - 1) // CH

    @functools.partial(
        pl.kernel,
        mesh=_sc_mesh(),
        out_type=jax.ShapeDtypeStruct((NC, n_pad, dim), jnp.float32),
        scratch_types=[
            pltpu.VMEM((CH,), jnp.int32),
            pltpu.VMEM((CH,), jnp.int32),
            pltpu.VMEM((CH,), jnp.int32),
            pltpu.VMEM((CH,), jnp.int32),
            pltpu.VMEM((CH, dim), jnp.float32),
            pltpu.VMEM((CH, dim), jnp.float32),
            pltpu.VMEM_SHARED((n_pad, dim), jnp.float32),
            pltpu.SemaphoreType.DMA,
            pltpu.SemaphoreType.DMA,
            pltpu.SemaphoreType.DMA,
        ],
    )
    def agg_kernel(h_hbm, zeros_hbm, src_hbm, dst_hbm, out_hbm,
                   src_a, dst_a, src_b, dst_b, rows_a, rows_b, y_sh,
                   sem_a, sem_b, sem2):
        c = lax.axis_index("c")
        s = lax.axis_index("s")
        row0 = s * rows_per_tile

        @pl.when(c == 0)
        def _():
            pltpu.sync_copy(h_hbm.at[pl.ds(row0, rows_per_tile)],
                            y_sh.at[pl.ds(row0, rows_per_tile)])

        @pl.when(c != 0)
        def _():
            pltpu.sync_copy(zeros_hbm.at[pl.ds(row0, rows_per_tile)],
                            y_sh.at[pl.ds(row0, rows_per_tile)])

        plsc.subcore_barrier()

        def fetch_idx(off, sv, dv):
            pltpu.async_copy(src_hbm.at[pl.ds(off, CH)], sv, sem2)
            pltpu.async_copy(dst_hbm.at[pl.ds(off, CH)], dv, sem2)

        def wait_idx(off, sv, dv):
            pltpu.make_async_copy(src_hbm.at[pl.ds(off, CH)], sv, sem2).wait()
            pltpu.make_async_copy(dst_hbm.at[pl.ds(off, CH)], dv, sem2).wait()

        def edge_loop(base_chunk, cpt):
            base = base_chunk * CH
            fetch_idx(base, src_a, dst_a)
            fetch_idx(base + CH, src_b, dst_b)
            wait_idx(base, src_a, dst_a)
            pltpu.async_copy(h_hbm.at[src_a], rows_a, sem_a)

            @pl.loop(0, cpt // 2)
            def _(jj):
                off0 = base + jj * (2 * CH)
                wait_idx(off0 + CH, src_b, dst_b)
                pltpu.async_copy(h_hbm.at[src_b], rows_b, sem_b)
                pltpu.make_async_copy(h_hbm.at[src_a], rows_a, sem_a).wait()
                pltpu.sync_copy(rows_a, y_sh.at[dst_a], add=True)
                fetch_idx(off0 + 2 * CH, src_a, dst_a)
                wait_idx(off0 + 2 * CH, src_a, dst_a)
                pltpu.async_copy(h_hbm.at[src_a], rows_a, sem_a)
                pltpu.make_async_copy(h_hbm.at[src_b], rows_b, sem_b).wait()
                pltpu.sync_copy(rows_b, y_sh.at[dst_b], add=True)
                fetch_idx(off0 + 3 * CH, src_b, dst_b)

            end = base + cpt * CH
            pltpu.make_async_copy(h_hbm.at[src_a], rows_a, sem_a).wait()
            wait_idx(end + CH, src_b, dst_b)

        @pl.when(c == 0)
        def _():
            edge_loop(s * cpt0, cpt0)

        @pl.when(c != 0)
        def _():
            edge_loop(NS * cpt0 + s * cpt1, cpt1)

        plsc.subcore_barrier()

        pltpu.sync_copy(y_sh.at[pl.ds(row0, rows_per_tile)],
                        out_hbm.at[c, pl.ds(row0, rows_per_tile), :])

    return agg_kernel


def _mm_scale_body(n, fts_ref, w_ref, deg_ref, out_ref):
    dinv = lax.rsqrt(deg_ref[...])
    h = jnp.dot(fts_ref[...], w_ref[...],
                preferred_element_type=jnp.float32) * dinv
    out_ref[:n] = h


def _mid_body(n, y_ref, deg_ref, b_ref, w_ref, out_ref):
    dinv = lax.rsqrt(deg_ref[...])
    ysum = (y_ref[0, :n] + y_ref[1, :n]) * dinv
    x = jnp.maximum(ysum + b_ref[...], 0.0)
    out_ref[:n] = jnp.dot(x, w_ref[...],
                          preferred_element_type=jnp.float32) * dinv


def _final_body(n, y_ref, deg_ref, b_ref, wc_ref, bc_ref, out_ref, hid_ref):
    dinv = lax.rsqrt(deg_ref[...])
    ysum = (y_ref[0, :n] + y_ref[1, :n]) * dinv
    x = jnp.maximum(ysum + b_ref[...], 0.0)
    hid_ref[...] = x
    out_ref[...] = jnp.dot(x, wc_ref[...],
                           preferred_element_type=jnp.float32) + bc_ref[...]


def kernel(fts, edge_index, W1, b1, W2, b2, Wc, bc):
    n, in_dim = fts.shape
    hid_dim = W1.shape[1]
    out_dim = Wc.shape[1]
    e = edge_index.shape[1]

    n_pad = ((n + 1 + NS * 8 - 1) // (NS * 8)) * (NS * 8)
    e_quant = NC * NS * CH
    e_pad = ((e + e_quant - 1) // e_quant) * e_quant

    src = edge_index[0]
    dst = edge_index[1]
    pad = e_pad + 2 * CH - e
    src_p = jnp.concatenate([src, jnp.zeros((pad,), jnp.int32)])
    pad_dst = n + jnp.arange(pad, dtype=jnp.int32) % (n_pad - n)
    dst_p = jnp.concatenate([dst, pad_dst])
    ones_pad = jnp.ones((n_pad,), jnp.float32)
    zeros_rows = jnp.zeros((n_pad, hid_dim), jnp.float32)

    deg_kernel = _make_deg_kernel(n_pad, e_pad)
    agg_kernel = _make_agg_kernel(n_pad, e_pad, hid_dim)

    zeros_1d = jnp.zeros((n_pad,), jnp.float32)
    deg_full = deg_kernel(dst_p, ones_pad, zeros_1d)
    deg = (deg_full[:n] + deg_full[n_pad:n_pad + n]).reshape(n, 1)

    b1r = b1.reshape(1, hid_dim)
    b2r = b2.reshape(1, hid_dim)
    bcr = bc.reshape(1, out_dim)

    h1 = pl.pallas_call(
        functools.partial(_mm_scale_body, n),
        out_shape=jax.ShapeDtypeStruct((n_pad, hid_dim), jnp.float32),
    )(fts, W1, deg)

    y1 = agg_kernel(h1, zeros_rows, src_p, dst_p)

    h2 = pl.pallas_call(
        functools.partial(_mid_body, n),
        out_shape=jax.ShapeDtypeStruct((n_pad, hid_dim), jnp.float32),
    )(y1, deg, b1r, W2)

    y2 = agg_kernel(h2, zeros_rows, src_p, dst_p)

    out, hid = pl.pallas_call(
        functools.partial(_final_body, n),
        out_shape=(
            jax.ShapeDtypeStruct((n, out_dim), jnp.float32),
            jax.ShapeDtypeStruct((n, hid_dim), jnp.float32),
        ),
    )(y2, deg, b2r, Wc, bcr)

    return (out, hid)


# speedup vs baseline: 1.0123x; 1.0123x over previous
import functools

import jax
import jax.numpy as jnp
from jax import lax
from jax.experimental import pallas as pl
from jax.experimental.pallas import tpu as pltpu
from jax.experimental.pallas import tpu_sc as plsc

NC = 2
NS = 16
CH = 128


def _sc_mesh():
    return plsc.VectorSubcoreMesh(core_axis_name="c", subcore_axis_name="s")


def _make_deg_kernel(n_pad, e_pad):
    chunks_per_tile = e_pad // (NC * NS * CH)
    rows_per_tile = n_pad // NS

    @functools.partial(
        pl.kernel,
        mesh=_sc_mesh(),
        out_type=jax.ShapeDtypeStruct((NC * n_pad,), jnp.float32),
        scratch_types=[
            pltpu.VMEM((CH,), jnp.int32),
            pltpu.VMEM((CH,), jnp.float32),
            pltpu.VMEM((rows_per_tile,), jnp.float32),
            pltpu.VMEM_SHARED((n_pad,), jnp.float32),
            pltpu.SemaphoreType.DMA,
        ],
    )
    def deg_kernel(dst_hbm, ones_hbm, zeros_hbm, out_hbm,
                   dst_v, ones_v, row_v, deg_sh, sem):
        c = lax.axis_index("c")
        s = lax.axis_index("s")
        row0 = s * rows_per_tile

        @pl.when(c == 0)
        def _():
            pltpu.sync_copy(ones_hbm.at[pl.ds(row0, rows_per_tile)], row_v)

        @pl.when(c != 0)
        def _():
            pltpu.sync_copy(zeros_hbm.at[pl.ds(row0, rows_per_tile)], row_v)

        pltpu.sync_copy(row_v, deg_sh.at[pl.ds(row0, rows_per_tile)])
        pltpu.sync_copy(ones_hbm.at[pl.ds(0, CH)], ones_v)
        plsc.subcore_barrier()

        base_chunk = (c * NS + s) * chunks_per_tile

        @pl.loop(0, chunks_per_tile)
        def _(j):
            off = (base_chunk + j) * CH
            pltpu.sync_copy(dst_hbm.at[pl.ds(off, CH)], dst_v)
            pltpu.sync_copy(ones_v, deg_sh.at[dst_v], add=True)

        plsc.subcore_barrier()

        pltpu.sync_copy(deg_sh.at[pl.ds(row0, rows_per_tile)], row_v)
        pltpu.sync_copy(row_v, out_hbm.at[pl.ds(c * n_pad + row0, rows_per_tile)])

    return deg_kernel


IDX_Q = 8


CPT_FRAC0 = 96 / 158


def _make_agg_kernel(n_pad, e_pad, dim):
    chunks_total = e_pad // (NS * CH)
    cpt0 = 2 * int(round(chunks_total * CPT_FRAC0 / 2))
    cpt1 = chunks_total - cpt0
    assert cpt1 % 2 == 0 and cpt0 >= 2 and cpt1 >= 2
    rows_per_tile = n_pad // NS
    n_init_chunks = (rows_per_tile + CH - 1) // CH

    @functools.partial(
        pl.kernel,
        mesh=_sc_mesh(),
        out_type=jax.ShapeDtypeStruct((NC, n_pad, dim), jnp.float32),
        scratch_types=[
            pltpu.VMEM((CH,), jnp.int32),
            pltpu.VMEM((CH,), jnp.int32),
            pltpu.VMEM((CH,), jnp.int32),
            pltpu.VMEM((CH,), jnp.int32),
            pltpu.VMEM((CH, dim), jnp.float32),
            pltpu.VMEM((CH, dim), jnp.float32),
            pltpu.VMEM_SHARED((n_pad, dim), jnp.float32),
            pltpu.SemaphoreType.DMA,
            pltpu.SemaphoreType.DMA,
            pltpu.SemaphoreType.DMA,
        ],
    )
    def agg_kernel(h_hbm, zeros_hbm, src_hbm, dst_hbm, out_hbm,
                   src_a, dst_a, src_b, dst_b, rows_a, rows_b, y_sh,
                   sem_a, sem_b, sem2):
        c = lax.axis_index("c")
        s = lax.axis_index("s")
        row0 = s * rows_per_tile

        @pl.when(c == 0)
        def _():
            pltpu.sync_copy(h_hbm.at[pl.ds(row0, rows_per_tile)],
                            y_sh.at[pl.ds(row0, rows_per_tile)])

        @pl.when(c != 0)
        def _():
            pltpu.sync_copy(zeros_hbm.at[pl.ds(row0, rows_per_tile)],
                            y_sh.at[pl.ds(row0, rows_per_tile)])

        plsc.subcore_barrier()

        def fetch_idx(off, sv, dv):
            pltpu.async_copy(src_hbm.at[pl.ds(off, CH)], sv, sem2)
            pltpu.async_copy(dst_hbm.at[pl.ds(off, CH)], dv, sem2)

        def wait_idx(off, sv, dv):
            pltpu.make_async_copy(src_hbm.at[pl.ds(off, CH)], sv, sem2).wait()
            pltpu.make_async_copy(dst_hbm.at[pl.ds(off, CH)], dv, sem2).wait()

        def edge_loop(base_chunk, cpt):
            base = base_chunk * CH
            fetch_idx(base, src_a, dst_a)
            fetch_idx(base + CH, src_b, dst_b)
            wait_idx(base, src_a, dst_a)
            pltpu.async_copy(h_hbm.at[src_a], rows_a, sem_a)

            @pl.loop(0, cpt // 2)
            def _(jj):
                off0 = base + jj * (2 * CH)
                wait_idx(off0 + CH, src_b, dst_b)
                pltpu.async_copy(h_hbm.at[src_b], rows_b, sem_b)
                pltpu.make_async_copy(h_hbm.at[src_a], rows_a, sem_a).wait()
                pltpu.sync_copy(rows_a, y_sh.at[dst_a], add=True)
                fetch_idx(off0 + 2 * CH, src_a, dst_a)
                wait_idx(off0 + 2 * CH, src_a, dst_a)
                pltpu.async_copy(h_hbm.at[src_a], rows_a, sem_a)
                pltpu.make_async_copy(h_hbm.at[src_b], rows_b, sem_b).wait()
                pltpu.sync_copy(rows_b, y_sh.at[dst_b], add=True)
                fetch_idx(off0 + 3 * CH, src_b, dst_b)

            end = base + cpt * CH
            pltpu.make_async_copy(h_hbm.at[src_a], rows_a, sem_a).wait()
            wait_idx(end + CH, src_b, dst_b)

        @pl.when(c == 0)
        def _():
            edge_loop(s * cpt0, cpt0)

        @pl.when(c != 0)
        def _():
            edge_loop(NS * cpt0 + s * cpt1, cpt1)

        plsc.subcore_barrier()

        pltpu.sync_copy(y_sh.at[pl.ds(row0, rows_per_tile)],
                        out_hbm.at[c, pl.ds(row0, rows_per_tile), :])

    return agg_kernel


def _mm_scale_body(n, fts_ref, w_ref, deg_ref, out_ref):
    dinv = lax.rsqrt(deg_ref[...])
    h = jnp.dot(fts_ref[...], w_ref[...],
                preferred_element_type=jnp.float32) * dinv
    out_ref[:n] = h


def _mid_body(n, y_ref, deg_ref, b_ref, w_ref, out_ref):
    dinv = lax.rsqrt(deg_ref[...])
    ysum = (y_ref[0, :n] + y_ref[1, :n]) * dinv
    x = jnp.maximum(ysum + b_ref[...], 0.0)
    out_ref[:n] = jnp.dot(x, w_ref[...],
                          preferred_element_type=jnp.float32) * dinv


def _final_body(n, y_ref, deg_ref, b_ref, wc_ref, bc_ref, out_ref, hid_ref):
    dinv = lax.rsqrt(deg_ref[...])
    ysum = (y_ref[0, :n] + y_ref[1, :n]) * dinv
    x = jnp.maximum(ysum + b_ref[...], 0.0)
    hid_ref[...] = x
    out_ref[...] = jnp.dot(x, wc_ref[...],
                           preferred_element_type=jnp.float32) + bc_ref[...]


def kernel(fts, edge_index, W1, b1, W2, b2, Wc, bc):
    n, in_dim = fts.shape
    hid_dim = W1.shape[1]
    out_dim = Wc.shape[1]
    e = edge_index.shape[1]

    n_pad = ((n + 1 + NS * 8 - 1) // (NS * 8)) * (NS * 8)
    e_quant = NC * NS * CH
    e_pad = ((e + e_quant - 1) // e_quant) * e_quant

    src = edge_index[0]
    dst = edge_index[1]
    pad = e_pad + 2 * CH - e
    src_p = jnp.concatenate([src, jnp.zeros((pad,), jnp.int32)])
    pad_dst = n + jnp.arange(pad, dtype=jnp.int32) % (n_pad - n)
    dst_p = jnp.concatenate([dst, pad_dst])
    ones_pad = jnp.ones((n_pad,), jnp.float32)
    zeros_rows = jnp.zeros((n_pad, hid_dim), jnp.float32)

    deg_kernel = _make_deg_kernel(n_pad, e_pad)
    agg_kernel = _make_agg_kernel(n_pad, e_pad, hid_dim)

    zeros_1d = jnp.zeros((n_pad,), jnp.float32)
    deg_full = deg_kernel(dst_p, ones_pad, zeros_1d)
    deg = (deg_full[:n] + deg_full[n_pad:n_pad + n]).reshape(n, 1)

    b1r = b1.reshape(1, hid_dim)
    b2r = b2.reshape(1, hid_dim)
    bcr = bc.reshape(1, out_dim)

    h1 = pl.pallas_call(
        functools.partial(_mm_scale_body, n),
        out_shape=jax.ShapeDtypeStruct((n_pad, hid_dim), jnp.float32),
    )(fts, W1, deg)

    y1 = agg_kernel(h1, zeros_rows, src_p, dst_p)

    h2 = pl.pallas_call(
        functools.partial(_mid_body, n),
        out_shape=jax.ShapeDtypeStruct((n_pad, hid_dim), jnp.float32),
    )(y1, deg, b1r, W2)

    y2 = agg_kernel(h2, zeros_rows, src_p, dst_p)

    out, hid = pl.pallas_call(
        functools.partial(_final_body, n),
        out_shape=(
            jax.ShapeDtypeStruct((n, out_dim), jnp.float32),
            jax.ShapeDtypeStruct((n, hid_dim), jnp.float32),
        ),
    )(y2, deg, b2r, Wc, bcr)

    return (out, hid)


# speedup vs baseline: 1.0229x; 1.0104x over previous
import functools

import jax
import jax.numpy as jnp
from jax import lax
from jax.experimental import pallas as pl
from jax.experimental.pallas import tpu as pltpu
from jax.experimental.pallas import tpu_sc as plsc

NC = 2
NS = 16
CH = 128


def _sc_mesh():
    return plsc.VectorSubcoreMesh(core_axis_name="c", subcore_axis_name="s")


def _make_deg_kernel(n_pad, e_pad):
    chunks_per_tile = e_pad // (NC * NS * CH)
    rows_per_tile = n_pad // NS

    @functools.partial(
        pl.kernel,
        mesh=_sc_mesh(),
        out_type=jax.ShapeDtypeStruct((NC * n_pad,), jnp.float32),
        scratch_types=[
            pltpu.VMEM((CH,), jnp.int32),
            pltpu.VMEM((CH,), jnp.float32),
            pltpu.VMEM((rows_per_tile,), jnp.float32),
            pltpu.VMEM_SHARED((n_pad,), jnp.float32),
            pltpu.SemaphoreType.DMA,
        ],
    )
    def deg_kernel(dst_hbm, ones_hbm, zeros_hbm, out_hbm,
                   dst_v, ones_v, row_v, deg_sh, sem):
        c = lax.axis_index("c")
        s = lax.axis_index("s")
        row0 = s * rows_per_tile

        @pl.when(c == 0)
        def _():
            pltpu.sync_copy(ones_hbm.at[pl.ds(row0, rows_per_tile)], row_v)

        @pl.when(c != 0)
        def _():
            pltpu.sync_copy(zeros_hbm.at[pl.ds(row0, rows_per_tile)], row_v)

        pltpu.sync_copy(row_v, deg_sh.at[pl.ds(row0, rows_per_tile)])
        pltpu.sync_copy(ones_hbm.at[pl.ds(0, CH)], ones_v)
        plsc.subcore_barrier()

        base_chunk = (c * NS + s) * chunks_per_tile

        @pl.loop(0, chunks_per_tile)
        def _(j):
            off = (base_chunk + j) * CH
            pltpu.sync_copy(dst_hbm.at[pl.ds(off, CH)], dst_v)
            pltpu.sync_copy(ones_v, deg_sh.at[dst_v], add=True)

        plsc.subcore_barrier()

        pltpu.sync_copy(deg_sh.at[pl.ds(row0, rows_per_tile)], row_v)
        pltpu.sync_copy(row_v, out_hbm.at[pl.ds(c * n_pad + row0, rows_per_tile)])

    return deg_kernel


IDX_Q = 8


CPT_FRAC0 = 100 / 158


def _make_agg_kernel(n_pad, e_pad, dim):
    chunks_total = e_pad // (NS * CH)
    cpt0 = 2 * int(round(chunks_total * CPT_FRAC0 / 2))
    cpt1 = chunks_total - cpt0
    assert cpt1 % 2 == 0 and cpt0 >= 2 and cpt1 >= 2
    rows_per_tile = n_pad // NS
    n_init_chunks = (rows_per_tile + CH - 1) // CH

    @functools.partial(
        pl.kernel,
        mesh=_sc_mesh(),
        out_type=jax.ShapeDtypeStruct((NC, n_pad, dim), jnp.float32),
        scratch_types=[
            pltpu.VMEM((CH,), jnp.int32),
            pltpu.VMEM((CH,), jnp.int32),
            pltpu.VMEM((CH,), jnp.int32),
            pltpu.VMEM((CH,), jnp.int32),
            pltpu.VMEM((CH, dim), jnp.float32),
            pltpu.VMEM((CH, dim), jnp.float32),
            pltpu.VMEM_SHARED((n_pad, dim), jnp.float32),
            pltpu.SemaphoreType.DMA,
            pltpu.SemaphoreType.DMA,
            pltpu.SemaphoreType.DMA,
        ],
    )
    def agg_kernel(h_hbm, zeros_hbm, src_hbm, dst_hbm, out_hbm,
                   src_a, dst_a, src_b, dst_b, rows_a, rows_b, y_sh,
                   sem_a, sem_b, sem2):
        c = lax.axis_index("c")
        s = lax.axis_index("s")
        row0 = s * rows_per_tile

        @pl.when(c == 0)
        def _():
            pltpu.sync_copy(h_hbm.at[pl.ds(row0, rows_per_tile)],
                            y_sh.at[pl.ds(row0, rows_per_tile)])

        @pl.when(c != 0)
        def _():
            pltpu.sync_copy(zeros_hbm.at[pl.ds(row0, rows_per_tile)],
                            y_sh.at[pl.ds(row0, rows_per_tile)])

        plsc.subcore_barrier()

        def fetch_idx(off, sv, dv):
            pltpu.async_copy(src_hbm.at[pl.ds(off, CH)], sv, sem2)
            pltpu.async_copy(dst_hbm.at[pl.ds(off, CH)], dv, sem2)

        def wait_idx(off, sv, dv):
            pltpu.make_async_copy(src_hbm.at[pl.ds(off, CH)], sv, sem2).wait()
            pltpu.make_async_copy(dst_hbm.at[pl.ds(off, CH)], dv, sem2).wait()

        def edge_loop(base_chunk, cpt):
            base = base_chunk * CH
            fetch_idx(base, src_a, dst_a)
            fetch_idx(base + CH, src_b, dst_b)
            wait_idx(base, src_a, dst_a)
            pltpu.async_copy(h_hbm.at[src_a], rows_a, sem_a)

            @pl.loop(0, cpt // 2)
            def _(jj):
                off0 = base + jj * (2 * CH)
                wait_idx(off0 + CH, src_b, dst_b)
                pltpu.async_copy(h_hbm.at[src_b], rows_b, sem_b)
                pltpu.make_async_copy(h_hbm.at[src_a], rows_a, sem_a).wait()
                pltpu.sync_copy(rows_a, y_sh.at[dst_a], add=True)
                fetch_idx(off0 + 2 * CH, src_a, dst_a)
                wait_idx(off0 + 2 * CH, src_a, dst_a)
                pltpu.async_copy(h_hbm.at[src_a], rows_a, sem_a)
                pltpu.make_async_copy(h_hbm.at[src_b], rows_b, sem_b).wait()
                pltpu.sync_copy(rows_b, y_sh.at[dst_b], add=True)
                fetch_idx(off0 + 3 * CH, src_b, dst_b)

            end = base + cpt * CH
            pltpu.make_async_copy(h_hbm.at[src_a], rows_a, sem_a).wait()
            wait_idx(end + CH, src_b, dst_b)

        @pl.when(c == 0)
        def _():
            edge_loop(s * cpt0, cpt0)

        @pl.when(c != 0)
        def _():
            edge_loop(NS * cpt0 + s * cpt1, cpt1)

        plsc.subcore_barrier()

        pltpu.sync_copy(y_sh.at[pl.ds(row0, rows_per_tile)],
                        out_hbm.at[c, pl.ds(row0, rows_per_tile), :])

    return agg_kernel


def _mm_scale_body(n, fts_ref, w_ref, deg_ref, out_ref):
    dinv = lax.rsqrt(deg_ref[...])
    h = jnp.dot(fts_ref[...], w_ref[...],
                preferred_element_type=jnp.float32) * dinv
    out_ref[:n] = h


def _mid_body(n, y_ref, deg_ref, b_ref, w_ref, out_ref):
    dinv = lax.rsqrt(deg_ref[...])
    ysum = (y_ref[0, :n] + y_ref[1, :n]) * dinv
    x = jnp.maximum(ysum + b_ref[...], 0.0)
    out_ref[:n] = jnp.dot(x, w_ref[...],
                          preferred_element_type=jnp.float32) * dinv


def _final_body(n, y_ref, deg_ref, b_ref, wc_ref, bc_ref, out_ref, hid_ref):
    dinv = lax.rsqrt(deg_ref[...])
    ysum = (y_ref[0, :n] + y_ref[1, :n]) * dinv
    x = jnp.maximum(ysum + b_ref[...], 0.0)
    hid_ref[...] = x
    out_ref[...] = jnp.dot(x, wc_ref[...],
                           preferred_element_type=jnp.float32) + bc_ref[...]


def kernel(fts, edge_index, W1, b1, W2, b2, Wc, bc):
    n, in_dim = fts.shape
    hid_dim = W1.shape[1]
    out_dim = Wc.shape[1]
    e = edge_index.shape[1]

    n_pad = ((n + 1 + NS * 8 - 1) // (NS * 8)) * (NS * 8)
    e_quant = NC * NS * CH
    e_pad = ((e + e_quant - 1) // e_quant) * e_quant

    src = edge_index[0]
    dst = edge_index[1]
    pad = e_pad + 2 * CH - e
    src_p = jnp.concatenate([src, jnp.zeros((pad,), jnp.int32)])
    pad_dst = n + jnp.arange(pad, dtype=jnp.int32) % (n_pad - n)
    dst_p = jnp.concatenate([dst, pad_dst])
    ones_pad = jnp.ones((n_pad,), jnp.float32)
    zeros_rows = jnp.zeros((n_pad, hid_dim), jnp.float32)

    deg_kernel = _make_deg_kernel(n_pad, e_pad)
    agg_kernel = _make_agg_kernel(n_pad, e_pad, hid_dim)

    zeros_1d = jnp.zeros((n_pad,), jnp.float32)
    deg_full = deg_kernel(dst_p, ones_pad, zeros_1d)
    deg = (deg_full[:n] + deg_full[n_pad:n_pad + n]).reshape(n, 1)

    b1r = b1.reshape(1, hid_dim)
    b2r = b2.reshape(1, hid_dim)
    bcr = bc.reshape(1, out_dim)

    h1 = pl.pallas_call(
        functools.partial(_mm_scale_body, n),
        out_shape=jax.ShapeDtypeStruct((n_pad, hid_dim), jnp.float32),
    )(fts, W1, deg)

    y1 = agg_kernel(h1, zeros_rows, src_p, dst_p)

    h2 = pl.pallas_call(
        functools.partial(_mid_body, n),
        out_shape=jax.ShapeDtypeStruct((n_pad, hid_dim), jnp.float32),
    )(y1, deg, b1r, W2)

    y2 = agg_kernel(h2, zeros_rows, src_p, dst_p)

    out, hid = pl.pallas_call(
        functools.partial(_final_body, n),
        out_shape=(
            jax.ShapeDtypeStruct((n, out_dim), jnp.float32),
            jax.ShapeDtypeStruct((n, hid_dim), jnp.float32),
        ),
    )(y2, deg, b2r, Wc, bcr)

    return (out, hid)


# speedup vs baseline: 1.0295x; 1.0065x over previous
import functools

import jax
import jax.numpy as jnp
from jax import lax
from jax.experimental import pallas as pl
from jax.experimental.pallas import tpu as pltpu
from jax.experimental.pallas import tpu_sc as plsc

NC = 2
NS = 16
CH = 128


def _sc_mesh():
    return plsc.VectorSubcoreMesh(core_axis_name="c", subcore_axis_name="s")


def _make_deg_kernel(n_pad, e_pad):
    chunks_per_tile = e_pad // (NC * NS * CH)
    rows_per_tile = n_pad // NS

    @functools.partial(
        pl.kernel,
        mesh=_sc_mesh(),
        out_type=jax.ShapeDtypeStruct((NC * n_pad,), jnp.float32),
        scratch_types=[
            pltpu.VMEM((CH,), jnp.int32),
            pltpu.VMEM((CH,), jnp.float32),
            pltpu.VMEM((rows_per_tile,), jnp.float32),
            pltpu.VMEM_SHARED((n_pad,), jnp.float32),
            pltpu.SemaphoreType.DMA,
        ],
    )
    def deg_kernel(dst_hbm, ones_hbm, zeros_hbm, out_hbm,
                   dst_v, ones_v, row_v, deg_sh, sem):
        c = lax.axis_index("c")
        s = lax.axis_index("s")
        row0 = s * rows_per_tile

        @pl.when(c == 0)
        def _():
            pltpu.sync_copy(ones_hbm.at[pl.ds(row0, rows_per_tile)], row_v)

        @pl.when(c != 0)
        def _():
            pltpu.sync_copy(zeros_hbm.at[pl.ds(row0, rows_per_tile)], row_v)

        pltpu.sync_copy(row_v, deg_sh.at[pl.ds(row0, rows_per_tile)])
        pltpu.sync_copy(ones_hbm.at[pl.ds(0, CH)], ones_v)
        plsc.subcore_barrier()

        base_chunk = (c * NS + s) * chunks_per_tile

        @pl.loop(0, chunks_per_tile)
        def _(j):
            off = (base_chunk + j) * CH
            pltpu.sync_copy(dst_hbm.at[pl.ds(off, CH)], dst_v)
            pltpu.sync_copy(ones_v, deg_sh.at[dst_v], add=True)

        plsc.subcore_barrier()

        pltpu.sync_copy(deg_sh.at[pl.ds(row0, rows_per_tile)], row_v)
        pltpu.sync_copy(row_v, out_hbm.at[pl.ds(c * n_pad + row0, rows_per_tile)])

    return deg_kernel


IDX_Q = 8


CPT_FRAC0 = 104 / 158


def _make_agg_kernel(n_pad, e_pad, dim):
    chunks_total = e_pad // (NS * CH)
    cpt0 = 2 * int(round(chunks_total * CPT_FRAC0 / 2))
    cpt1 = chunks_total - cpt0
    assert cpt1 % 2 == 0 and cpt0 >= 2 and cpt1 >= 2
    rows_per_tile = n_pad // NS
    n_init_chunks = (rows_per_tile + CH - 1) // CH

    @functools.partial(
        pl.kernel,
        mesh=_sc_mesh(),
        out_type=jax.ShapeDtypeStruct((NC, n_pad, dim), jnp.float32),
        scratch_types=[
            pltpu.VMEM((CH,), jnp.int32),
            pltpu.VMEM((CH,), jnp.int32),
            pltpu.VMEM((CH,), jnp.int32),
            pltpu.VMEM((CH,), jnp.int32),
            pltpu.VMEM((CH, dim), jnp.float32),
            pltpu.VMEM((CH, dim), jnp.float32),
            pltpu.VMEM_SHARED((n_pad, dim), jnp.float32),
            pltpu.SemaphoreType.DMA,
            pltpu.SemaphoreType.DMA,
            pltpu.SemaphoreType.DMA,
        ],
    )
    def agg_kernel(h_hbm, zeros_hbm, src_hbm, dst_hbm, out_hbm,
                   src_a, dst_a, src_b, dst_b, rows_a, rows_b, y_sh,
                   sem_a, sem_b, sem2):
        c = lax.axis_index("c")
        s = lax.axis_index("s")
        row0 = s * rows_per_tile

        @pl.when(c == 0)
        def _():
            pltpu.sync_copy(h_hbm.at[pl.ds(row0, rows_per_tile)],
                            y_sh.at[pl.ds(row0, rows_per_tile)])

        @pl.when(c != 0)
        def _():
            pltpu.sync_copy(zeros_hbm.at[pl.ds(row0, rows_per_tile)],
                            y_sh.at[pl.ds(row0, rows_per_tile)])

        plsc.subcore_barrier()

        def fetch_idx(off, sv, dv):
            pltpu.async_copy(src_hbm.at[pl.ds(off, CH)], sv, sem2)
            pltpu.async_copy(dst_hbm.at[pl.ds(off, CH)], dv, sem2)

        def wait_idx(off, sv, dv):
            pltpu.make_async_copy(src_hbm.at[pl.ds(off, CH)], sv, sem2).wait()
            pltpu.make_async_copy(dst_hbm.at[pl.ds(off, CH)], dv, sem2).wait()

        def edge_loop(base_chunk, cpt):
            base = base_chunk * CH
            fetch_idx(base, src_a, dst_a)
            fetch_idx(base + CH, src_b, dst_b)
            wait_idx(base, src_a, dst_a)
            pltpu.async_copy(h_hbm.at[src_a], rows_a, sem_a)

            @pl.loop(0, cpt // 2)
            def _(jj):
                off0 = base + jj * (2 * CH)
                wait_idx(off0 + CH, src_b, dst_b)
                pltpu.async_copy(h_hbm.at[src_b], rows_b, sem_b)
                pltpu.make_async_copy(h_hbm.at[src_a], rows_a, sem_a).wait()
                pltpu.sync_copy(rows_a, y_sh.at[dst_a], add=True)
                fetch_idx(off0 + 2 * CH, src_a, dst_a)
                wait_idx(off0 + 2 * CH, src_a, dst_a)
                pltpu.async_copy(h_hbm.at[src_a], rows_a, sem_a)
                pltpu.make_async_copy(h_hbm.at[src_b], rows_b, sem_b).wait()
                pltpu.sync_copy(rows_b, y_sh.at[dst_b], add=True)
                fetch_idx(off0 + 3 * CH, src_b, dst_b)

            end = base + cpt * CH
            pltpu.make_async_copy(h_hbm.at[src_a], rows_a, sem_a).wait()
            wait_idx(end + CH, src_b, dst_b)

        @pl.when(c == 0)
        def _():
            edge_loop(s * cpt0, cpt0)

        @pl.when(c != 0)
        def _():
            edge_loop(NS * cpt0 + s * cpt1, cpt1)

        plsc.subcore_barrier()

        pltpu.sync_copy(y_sh.at[pl.ds(row0, rows_per_tile)],
                        out_hbm.at[c, pl.ds(row0, rows_per_tile), :])

    return agg_kernel


def _mm_scale_body(n, fts_ref, w_ref, deg_ref, out_ref):
    dinv = lax.rsqrt(deg_ref[...])
    h = jnp.dot(fts_ref[...], w_ref[...],
                preferred_element_type=jnp.float32) * dinv
    out_ref[:n] = h


def _mid_body(n, y_ref, deg_ref, b_ref, w_ref, out_ref):
    dinv = lax.rsqrt(deg_ref[...])
    ysum = (y_ref[0, :n] + y_ref[1, :n]) * dinv
    x = jnp.maximum(ysum + b_ref[...], 0.0)
    out_ref[:n] = jnp.dot(x, w_ref[...],
                          preferred_element_type=jnp.float32) * dinv


def _final_body(n, y_ref, deg_ref, b_ref, wc_ref, bc_ref, out_ref, hid_ref):
    dinv = lax.rsqrt(deg_ref[...])
    ysum = (y_ref[0, :n] + y_ref[1, :n]) * dinv
    x = jnp.maximum(ysum + b_ref[...], 0.0)
    hid_ref[...] = x
    out_ref[...] = jnp.dot(x, wc_ref[...],
                           preferred_element_type=jnp.float32) + bc_ref[...]


def kernel(fts, edge_index, W1, b1, W2, b2, Wc, bc):
    n, in_dim = fts.shape
    hid_dim = W1.shape[1]
    out_dim = Wc.shape[1]
    e = edge_index.shape[1]

    n_pad = ((n + 1 + NS * 8 - 1) // (NS * 8)) * (NS * 8)
    e_quant = NC * NS * CH
    e_pad = ((e + e_quant - 1) // e_quant) * e_quant

    src = edge_index[0]
    dst = edge_index[1]
    pad = e_pad + 2 * CH - e
    src_p = jnp.concatenate([src, jnp.zeros((pad,), jnp.int32)])
    pad_dst = n + jnp.arange(pad, dtype=jnp.int32) % (n_pad - n)
    dst_p = jnp.concatenate([dst, pad_dst])
    ones_pad = jnp.ones((n_pad,), jnp.float32)
    zeros_rows = jnp.zeros((n_pad, hid_dim), jnp.float32)

    deg_kernel = _make_deg_kernel(n_pad, e_pad)
    agg_kernel = _make_agg_kernel(n_pad, e_pad, hid_dim)

    zeros_1d = jnp.zeros((n_pad,), jnp.float32)
    deg_full = deg_kernel(dst_p, ones_pad, zeros_1d)
    deg = (deg_full[:n] + deg_full[n_pad:n_pad + n]).reshape(n, 1)

    b1r = b1.reshape(1, hid_dim)
    b2r = b2.reshape(1, hid_dim)
    bcr = bc.reshape(1, out_dim)

    h1 = pl.pallas_call(
        functools.partial(_mm_scale_body, n),
        out_shape=jax.ShapeDtypeStruct((n_pad, hid_dim), jnp.float32),
    )(fts, W1, deg)

    y1 = agg_kernel(h1, zeros_rows, src_p, dst_p)

    h2 = pl.pallas_call(
        functools.partial(_mid_body, n),
        out_shape=jax.ShapeDtypeStruct((n_pad, hid_dim), jnp.float32),
    )(y1, deg, b1r, W2)

    y2 = agg_kernel(h2, zeros_rows, src_p, dst_p)

    out, hid = pl.pallas_call(
        functools.partial(_final_body, n),
        out_shape=(
            jax.ShapeDtypeStruct((n, out_dim), jnp.float32),
            jax.ShapeDtypeStruct((n, hid_dim), jnp.float32),
        ),
    )(y2, deg, b2r, Wc, bcr)

    return (out, hid)
